# Initial kernel scaffold; baseline (speedup 1.0000x reference)
import jax
import jax.numpy as jnp
from jax.experimental import pallas as pl


def kernel(x, ori, bbox):
    raise NotImplementedError("write your pallas kernel here")


# speedup vs baseline: 1.2509x; 1.2509x over previous
import functools

import jax
import jax.numpy as jnp
from jax import lax
from jax.experimental import pallas as pl
from jax.experimental.pallas import tpu as pltpu
from jax.experimental.pallas import tpu_sc as plsc

_NC = 2
_NS = 16
_NW = _NC * _NS

_B, _C, _H, _W = 4, 96, 512, 512
_IMGS = (_B * _C) // _NW
_BLK = 32
_BPI = _H // _BLK
_NB = _IMGS * _BPI
_NBUF = 4
_LOOK = 2
_NV = _W // 16


def _body(xr, orir, params, out,
          buf0, buf1, buf2, buf3, xbuf, pv,
          is0, is1, is2, is3, os0, os1, os2, os3):
    bufs = (buf0, buf1, buf2, buf3)
    in_sems = (is0, is1, is2, is3)
    out_sems = (os0, os1, os2, os3)

    cid = lax.axis_index("c")
    sid = lax.axis_index("s")
    w = sid * _NC + cid
    row0 = w * (_IMGS * _H)

    pltpu.sync_copy(params, pv)

    def start_in(n, j):
        pltpu.make_async_copy(
            orir.at[pl.ds(row0 + n * _BLK, _BLK)], bufs[j], in_sems[j]).start()

    def wait_in(j):
        pltpu.make_async_copy(
            orir.at[pl.ds(0, _BLK)], bufs[j], in_sems[j]).wait()

    def start_out(n, j):
        pltpu.make_async_copy(
            bufs[j], out.at[pl.ds(row0 + n * _BLK, _BLK)], out_sems[j]).start()

    def wait_out(j):
        pltpu.make_async_copy(
            bufs[j], out.at[pl.ds(0, _BLK)], out_sems[j]).wait()

    def fixup(n, j):
        m = n // _BPI
        r0 = (n % _BPI) * _BLK
        g = w * _IMGS + m
        pvec = pv[pl.ds(g * 4, 16)]
        x1 = pvec[0]
        y1 = pvec[1]
        x2 = pvec[2]
        y2 = pvec[3]
        lo = jnp.maximum(y1, r0)
        hi = jnp.minimum(y2, r0 + _BLK)

        @pl.when((lo < hi) & (x1 < x2))
        def _():
            pltpu.sync_copy(xr.at[pl.ds(row0 + n * _BLK, _BLK)], xbuf)

            def rbody(r, carry):
                rr = r - r0
                for v in range(_NV):
                    cols = lax.iota(jnp.int32, 16) + (v * 16)
                    msk = (cols >= x1) & (cols < x2)
                    cur = bufs[j][rr, pl.ds(v * 16, 16)]
                    xv = xbuf[rr, pl.ds(v * 16, 16)]
                    bufs[j][rr, pl.ds(v * 16, 16)] = jnp.where(msk, xv, cur)
                return carry

            lax.fori_loop(lo, hi, rbody, 0)

    start_in(0, 0)
    start_in(1, 1)

    def outer(tt, carry):
        t = tt * _NBUF
        for j in range(_NBUF):
            n = t + j
            nxt = n + _LOOK
            s = (j + _LOOK) % _NBUF

            @pl.when(nxt < _NB)
            def _():
                @pl.when(nxt >= _NBUF)
                def _():
                    wait_out(s)
                start_in(nxt, s)

            wait_in(j)
            fixup(n, j)
            start_out(n, j)
        return carry

    lax.fori_loop(0, _NB // _NBUF, outer, 0)

    for j in range(_NBUF):
        wait_out(j)


@jax.jit
def kernel(x, ori, bbox):
    b, c, h, w = x.shape
    scale = jnp.array([h, w, h, w, 1.0], dtype=bbox.dtype)
    coords = (bbox * scale).astype(jnp.int32)
    per_img = jnp.repeat(coords[:, :4], c, axis=0)
    params = jnp.concatenate(
        [per_img.reshape(-1), jnp.zeros((16,), jnp.int32)])

    xr = x.reshape(b * c * h, w)
    orir = ori.reshape(b * c * h, w)

    run = pl.kernel(
        _body,
        out_type=jax.ShapeDtypeStruct((b * c * h, w), x.dtype),
        mesh=plsc.VectorSubcoreMesh(core_axis_name="c", subcore_axis_name="s"),
        scratch_types=(
            [pltpu.VMEM((_BLK, _W), jnp.float32)] * 4
            + [pltpu.VMEM((_BLK, _W), jnp.float32),
               pltpu.VMEM((_B * _C * 4 + 16,), jnp.int32)]
            + [pltpu.SemaphoreType.DMA] * 8
        ),
    )
    out = run(xr, orir, params)
    return out.reshape(b, c, h, w)
